# baseline (speedup 1.0000x reference)
import functools

import jax
import jax.numpy as jnp
import numpy as np
from jax import lax
from jax.experimental import pallas as pl
from jax.experimental.pallas import tpu as pltpu
from jax.experimental.pallas import tpu_sc as plsc

NBITS = 4
D = 512
NPAT = 16
NSEQ = 4096
SEQ = 50
NSC_SEQ = 2048
BSC = NSC_SEQ * SEQ
IDXR = BSC // 128
NREP = 32
R = 40
NBUF = 4
BLK = 32
ABLK = 64

_SEL = np.zeros((4 * 128, 128), dtype=np.float32)
for _c in range(128):
    for _b in range(NBITS):
        _SEL[4 * _c + _b, _c] = float(1 << _b)


def _tc1_body(e0_ref, e1_ref, w_ref, b_ref, g_ref, bb_ref, tok_ref, sel_ref,
              table_ref, idx_ref, dmat_ref, c0_ref):
    col = lax.broadcasted_iota(jnp.int32, (NPAT, D), 1)
    row = lax.broadcasted_iota(jnp.int32, (NPAT, D), 0)
    bit = ((row >> (col >> 7)) & 1).astype(jnp.float32)
    e0 = e0_ref[...]
    e1 = e1_ref[...]
    x = e0 + bit * (e1 - e0)
    y = lax.dot_general(x, w_ref[...], (((1,), (1,)), ((), ())),
                        preferred_element_type=jnp.float32)
    y = y + b_ref[...]
    mean = jnp.mean(y, axis=1, keepdims=True)
    var = jnp.mean(jnp.square(y - mean), axis=1, keepdims=True)
    yn = (y - mean) * lax.rsqrt(var + 1e-5)
    yn = yn * g_ref[...] + bb_ref[...]
    tab = yn * jax.nn.sigmoid(yn)
    table_ref[...] = jnp.broadcast_to(tab.reshape(1, NPAT, D),
                                      (NREP, NPAT, D))
    bcol = lax.broadcasted_iota(jnp.int32, (NBITS, D), 0)
    dcol = lax.broadcasted_iota(jnp.int32, (NBITS, D), 1)
    m = jnp.where((dcol >> 7) == bcol, (e1 - e0), 0.0)
    dmat_ref[...] = lax.dot_general(m, w_ref[...], (((1,), (1,)), ((), ())),
                                    preferred_element_type=jnp.float32)
    c0_ref[...] = lax.dot_general(e0, w_ref[...], (((1,), (1,)), ((), ())),
                                  preferred_element_type=jnp.float32) + b_ref[...]
    t = jnp.clip(tok_ref[...], 0, 1).astype(jnp.float32)
    idx_f = lax.dot_general(t, sel_ref[...], (((1,), (0,)), ((), ())),
                            preferred_element_type=jnp.float32)
    rr = lax.broadcasted_iota(jnp.int32, (IDXR, 128), 0)
    idx_ref[...] = idx_f.astype(jnp.int32) + (rr // 25) * NPAT


def _tc1_call(e0, e1, W, b2, g2, bb2, tok2, sel):
    return pl.pallas_call(
        _tc1_body,
        out_shape=(
            jax.ShapeDtypeStruct((NREP, NPAT, D), jnp.float32),
            jax.ShapeDtypeStruct((IDXR, 128), jnp.int32),
            jax.ShapeDtypeStruct((NBITS, D), jnp.float32),
            jax.ShapeDtypeStruct((1, D), jnp.float32),
        ),
    )(e0, e1, W, b2, g2, bb2, tok2, sel)


def _sc_call(table, idx_flat):
    info = plsc.get_sparse_core_info()
    nw = info.num_cores * info.num_subcores
    c_per = BSC // nw
    nch = c_per // R
    mesh = plsc.VectorSubcoreMesh(core_axis_name="c", subcore_axis_name="s")

    @functools.partial(
        pl.kernel,
        mesh=mesh,
        out_type=jax.ShapeDtypeStruct((BSC, D), jnp.float32),
        scratch_types=[
            pltpu.VMEM((c_per,), jnp.int32),
            pltpu.VMEM((NBUF, R, D), jnp.float32),
            pltpu.SemaphoreType.DMA,
            pltpu.SemaphoreType.DMA,
        ],
    )
    def k(table_hbm, idx_hbm, out_hbm, idx_v, rows_v, sem_g, sem_w):
        wid = lax.axis_index("s") * info.num_cores + lax.axis_index("c")
        base = wid * c_per
        pltpu.sync_copy(idx_hbm.at[pl.ds(base, c_per)], idx_v)

        def gather(ck):
            return pltpu.make_async_copy(
                table_hbm.at[idx_v.at[pl.ds(ck * R, R)]],
                rows_v.at[lax.rem(ck, NBUF)], sem_g)

        def write(ck):
            return pltpu.make_async_copy(
                rows_v.at[lax.rem(ck, NBUF)],
                out_hbm.at[pl.ds(base + ck * R, R)], sem_w)

        for j in range(NBUF - 1):
            gather(j).start()

        def body(ck, carry):
            gather(ck).wait()
            write(ck).start()
            nxt = ck + NBUF - 1

            @pl.when(nxt < nch)
            def _():
                @pl.when(nxt >= NBUF)
                def _():
                    write(nxt - NBUF).wait()
                gather(nxt).start()

            return carry

        lax.fori_loop(0, nch, body, 0)
        for j in range(max(0, nch - NBUF), nch):
            write(j).wait()

    return k(table, idx_flat)


def _dense_body(tok_ref, dmat_ref, c0_ref, g_ref, bb_ref, out_ref):
    t = jnp.clip(tok_ref[...], 0, 1).astype(jnp.float32)
    y = lax.dot_general(t, dmat_ref[...], (((2,), (0,)), ((), ())),
                        preferred_element_type=jnp.float32,
                        precision=lax.Precision.HIGHEST)
    y = y + c0_ref[...].reshape(1, 1, D)
    mean = jnp.mean(y, axis=2, keepdims=True)
    var = jnp.mean(jnp.square(y - mean), axis=2, keepdims=True)
    yn = (y - mean) * lax.rsqrt(var + 1e-5)
    yn = yn * g_ref[...].reshape(1, 1, D) + bb_ref[...].reshape(1, 1, D)
    out_ref[...] = yn * jax.nn.sigmoid(yn)


def _dense_call(tokens, dmat, c0, g2, bb2):
    n_dense = NSEQ
    grid = n_dense // BLK
    return pl.pallas_call(
        _dense_body,
        grid=(grid,),
        in_specs=[
            pl.BlockSpec((BLK, SEQ, NBITS),
                         lambda g: (g, 0, 0)),
            pl.BlockSpec((NBITS, D), lambda g: (0, 0)),
            pl.BlockSpec((1, D), lambda g: (0, 0)),
            pl.BlockSpec((1, D), lambda g: (0, 0)),
            pl.BlockSpec((1, D), lambda g: (0, 0)),
        ],
        out_specs=pl.BlockSpec((BLK, SEQ, D),
                               lambda g: (g, 0, 0)),
        out_shape=jax.ShapeDtypeStruct((NSEQ, SEQ, D), jnp.float32),
    )(tokens, dmat, c0, g2, bb2)


def _asm_body(temp_ref, buf_ref, out_ref):
    out_ref[...] = temp_ref[...].reshape(ABLK, SEQ, D)


def _asm_call(temp, buf):
    grid = NSC_SEQ // ABLK
    return pl.pallas_call(
        _asm_body,
        grid=(grid,),
        in_specs=[
            pl.BlockSpec((ABLK * SEQ, D), lambda g: (g, 0)),
            pl.BlockSpec(memory_space=pl.ANY),
        ],
        out_specs=pl.BlockSpec((ABLK, SEQ, D), lambda g: (g, 0, 0)),
        out_shape=jax.ShapeDtypeStruct((NSEQ, SEQ, D), jnp.float32),
        input_output_aliases={1: 0},
    )(temp, buf)


def kernel(tokens, emb, W, b, gamma, beta):
    tok2 = tokens[:NSC_SEQ].reshape(IDXR, 4 * 128)
    e0 = emb[:, 0, :].reshape(1, D)
    e1 = emb[:, 1, :].reshape(1, D)
    g2 = gamma.reshape(1, D)
    bb2 = beta.reshape(1, D)
    sel = jnp.asarray(_SEL)
    table, idx, dmat, c0 = _tc1_call(e0, e1, W, b.reshape(1, D), g2, bb2,
                                     tok2, sel)
    buf = _dense_call(tokens, dmat, c0, g2, bb2)
    return buf


# speedup vs baseline: 1.1557x; 1.1557x over previous
import functools

import jax
import jax.numpy as jnp
import numpy as np
from jax import lax
from jax.experimental import pallas as pl
from jax.experimental.pallas import tpu as pltpu
from jax.experimental.pallas import tpu_sc as plsc

NBITS = 4
D = 512
NPAT = 16
NSEQ = 4096
SEQ = 50
NSC_SEQ = 3072
BSC = NSC_SEQ * SEQ
IDXR = BSC // 128
NREP = 32
R = 40
NBUF = 4
BLK = 32
ABLK = 64

_SEL = np.zeros((4 * 128, 128), dtype=np.float32)
for _c in range(128):
    for _b in range(NBITS):
        _SEL[4 * _c + _b, _c] = float(1 << _b)


def _tc1_body(e0_ref, e1_ref, w_ref, b_ref, g_ref, bb_ref, tok_ref, sel_ref,
              table_ref, idx_ref, dmat_ref, c0_ref):
    col = lax.broadcasted_iota(jnp.int32, (NPAT, D), 1)
    row = lax.broadcasted_iota(jnp.int32, (NPAT, D), 0)
    bit = ((row >> (col >> 7)) & 1).astype(jnp.float32)
    e0 = e0_ref[...]
    e1 = e1_ref[...]
    x = e0 + bit * (e1 - e0)
    y = lax.dot_general(x, w_ref[...], (((1,), (1,)), ((), ())),
                        preferred_element_type=jnp.float32)
    y = y + b_ref[...]
    mean = jnp.mean(y, axis=1, keepdims=True)
    var = jnp.mean(jnp.square(y - mean), axis=1, keepdims=True)
    yn = (y - mean) * lax.rsqrt(var + 1e-5)
    yn = yn * g_ref[...] + bb_ref[...]
    tab = yn * jax.nn.sigmoid(yn)
    table_ref[...] = jnp.broadcast_to(tab.reshape(1, NPAT, D),
                                      (NREP, NPAT, D))
    bcol = lax.broadcasted_iota(jnp.int32, (NBITS, D), 0)
    dcol = lax.broadcasted_iota(jnp.int32, (NBITS, D), 1)
    m = jnp.where((dcol >> 7) == bcol, (e1 - e0), 0.0)
    dmat_ref[...] = lax.dot_general(m, w_ref[...], (((1,), (1,)), ((), ())),
                                    preferred_element_type=jnp.float32)
    c0_ref[...] = lax.dot_general(e0, w_ref[...], (((1,), (1,)), ((), ())),
                                  preferred_element_type=jnp.float32) + b_ref[...]
    t = jnp.clip(tok_ref[...], 0, 1).astype(jnp.float32)
    idx_f = lax.dot_general(t, sel_ref[...], (((1,), (0,)), ((), ())),
                            preferred_element_type=jnp.float32)
    rr = lax.broadcasted_iota(jnp.int32, (IDXR, 128), 0)
    idx_ref[...] = idx_f.astype(jnp.int32) + lax.rem(rr, NREP) * NPAT


def _tc1_call(e0, e1, W, b2, g2, bb2, tok2, sel):
    return pl.pallas_call(
        _tc1_body,
        out_shape=(
            jax.ShapeDtypeStruct((NREP, NPAT, D), jnp.float32),
            jax.ShapeDtypeStruct((IDXR, 128), jnp.int32),
            jax.ShapeDtypeStruct((NBITS, D), jnp.float32),
            jax.ShapeDtypeStruct((1, D), jnp.float32),
        ),
    )(e0, e1, W, b2, g2, bb2, tok2, sel)


def _sc_call(table, idx_flat):
    info = plsc.get_sparse_core_info()
    nw = info.num_cores * info.num_subcores
    c_per = BSC // nw
    nch = c_per // R
    mesh = plsc.VectorSubcoreMesh(core_axis_name="c", subcore_axis_name="s")

    @functools.partial(
        pl.kernel,
        mesh=mesh,
        out_type=jax.ShapeDtypeStruct((BSC, D), jnp.float32),
        scratch_types=[
            pltpu.VMEM((c_per,), jnp.int32),
            pltpu.VMEM((NBUF, R, D), jnp.float32),
            pltpu.SemaphoreType.DMA,
            pltpu.SemaphoreType.DMA,
        ],
    )
    def k(table_hbm, idx_hbm, out_hbm, idx_v, rows_v, sem_g, sem_w):
        wid = lax.axis_index("s") * info.num_cores + lax.axis_index("c")
        base = wid * c_per
        pltpu.sync_copy(idx_hbm.at[pl.ds(base, c_per)], idx_v)

        def gather(ck):
            return pltpu.make_async_copy(
                table_hbm.at[idx_v.at[pl.ds(ck * R, R)]],
                rows_v.at[lax.rem(ck, NBUF)], sem_g)

        def write(ck):
            return pltpu.make_async_copy(
                rows_v.at[lax.rem(ck, NBUF)],
                out_hbm.at[pl.ds(base + ck * R, R)], sem_w)

        for j in range(NBUF - 1):
            gather(j).start()

        def body(ck, carry):
            gather(ck).wait()
            write(ck).start()
            nxt = ck + NBUF - 1

            @pl.when(nxt < nch)
            def _():
                @pl.when(nxt >= NBUF)
                def _():
                    write(nxt - NBUF).wait()
                gather(nxt).start()

            return carry

        lax.fori_loop(0, nch, body, 0)
        for j in range(max(0, nch - NBUF), nch):
            write(j).wait()

    return k(table, idx_flat)


def _dense_body(tok_ref, dmat_ref, c0_ref, g_ref, bb_ref, out_ref):
    t = jnp.clip(tok_ref[...], 0, 1).astype(jnp.float32)
    y = lax.dot_general(t, dmat_ref[...], (((2,), (0,)), ((), ())),
                        preferred_element_type=jnp.float32)
    y = y + c0_ref[...].reshape(1, 1, D)
    mean = jnp.mean(y, axis=2, keepdims=True)
    var = jnp.mean(jnp.square(y - mean), axis=2, keepdims=True)
    yn = (y - mean) * lax.rsqrt(var + 1e-5)
    yn = yn * g_ref[...].reshape(1, 1, D) + bb_ref[...].reshape(1, 1, D)
    out_ref[...] = yn * jax.nn.sigmoid(yn)


def _dense_call(tokens, dmat, c0, g2, bb2):
    n_dense = NSEQ - NSC_SEQ
    grid = n_dense // BLK
    return pl.pallas_call(
        _dense_body,
        grid=(grid,),
        in_specs=[
            pl.BlockSpec((BLK, SEQ, NBITS),
                         lambda g: (NSC_SEQ // BLK + g, 0, 0)),
            pl.BlockSpec((NBITS, D), lambda g: (0, 0)),
            pl.BlockSpec((1, D), lambda g: (0, 0)),
            pl.BlockSpec((1, D), lambda g: (0, 0)),
            pl.BlockSpec((1, D), lambda g: (0, 0)),
        ],
        out_specs=pl.BlockSpec((BLK, SEQ, D),
                               lambda g: (NSC_SEQ // BLK + g, 0, 0)),
        out_shape=jax.ShapeDtypeStruct((NSEQ, SEQ, D), jnp.float32),
    )(tokens, dmat, c0, g2, bb2)


def _asm_body(temp_ref, buf_ref, out_ref):
    out_ref[...] = temp_ref[...].reshape(ABLK, SEQ, D)


def _asm_call(temp, buf):
    grid = NSC_SEQ // ABLK
    return pl.pallas_call(
        _asm_body,
        grid=(grid,),
        in_specs=[
            pl.BlockSpec((ABLK * SEQ, D), lambda g: (g, 0)),
            pl.BlockSpec(memory_space=pl.ANY),
        ],
        out_specs=pl.BlockSpec((ABLK, SEQ, D), lambda g: (g, 0, 0)),
        out_shape=jax.ShapeDtypeStruct((NSEQ, SEQ, D), jnp.float32),
        input_output_aliases={1: 0},
    )(temp, buf)


def kernel(tokens, emb, W, b, gamma, beta):
    tok2 = tokens[:NSC_SEQ].reshape(IDXR, 4 * 128)
    e0 = emb[:, 0, :].reshape(1, D)
    e1 = emb[:, 1, :].reshape(1, D)
    g2 = gamma.reshape(1, D)
    bb2 = beta.reshape(1, D)
    sel = jnp.asarray(_SEL)
    table, idx, dmat, c0 = _tc1_call(e0, e1, W, b.reshape(1, D), g2, bb2,
                                     tok2, sel)
    temp = _sc_call(table.reshape(NREP * NPAT, D), idx.reshape(-1))
    buf = _dense_call(tokens, dmat, c0, g2, bb2)
    return _asm_call(temp, buf)
